# baseline (speedup 1.0000x reference)
import jax
import jax.numpy as jnp
from jax import lax
from jax.experimental import pallas as pl
from jax.experimental.pallas import tpu as pltpu
from jax.experimental.pallas import tpu_sc as plsc

_MAX_SIZE = 50
_HALF = 128
_BATCH = 64
_ROWS = _MAX_SIZE * _MAX_SIZE
_NW = 32
_NU = 40


def _sc_body(row_hbm, col_hbm, out_hbm, rowv, colv, stage, sems):
    c = lax.axis_index("c")
    s = lax.axis_index("s")
    wid = s * 2 + c

    pltpu.sync_copy(row_hbm, rowv)
    pltpu.sync_copy(col_hbm, colv)

    def handle(t, p):
        r = jnp.minimum(wid + _NW * t, _ROWS - 1)
        i = r // _MAX_SIZE
        j = r - i * _MAX_SIZE

        for k in range(8):
            v = rowv[pl.ds(i * _HALF + k * 16, 16)]
            for rep in range(2):
                for sl in range(8):
                    stage[p, rep, 0, sl, pl.ds(k * 16, 16)] = v
        for k in range(8):
            v = colv[pl.ds(j * _HALF + k * 16, 16)]
            for rep in range(2):
                for sl in range(8):
                    stage[p, rep, 1, sl, pl.ds(k * 16, 16)] = v

        for st in range(4):
            pltpu.make_async_copy(
                stage.at[p],
                out_hbm.at[r, pl.ds(2 * st, 2)],
                sems.at[p],
            ).start()

    def drain(p):
        for st in range(4):
            pltpu.make_async_copy(
                stage.at[p], out_hbm.at[0, pl.ds(0, 2)], sems.at[p]
            ).wait()

    def body(u, carry):
        @pl.when(u >= 1)
        def _():
            drain(0)

        handle(2 * u, 0)

        @pl.when(u >= 1)
        def _():
            drain(1)

        handle(2 * u + 1, 1)
        return carry

    lax.fori_loop(0, _NU, body, 0)
    drain(0)
    drain(1)


def sc_kernel(batch_size, row_embed, col_embed):
    zero = (jnp.asarray(batch_size) - _BATCH).astype(row_embed.dtype)
    row_flat = (row_embed + zero).reshape(-1)
    col_flat = (col_embed + zero).reshape(-1)

    mesh = plsc.VectorSubcoreMesh(core_axis_name="c", subcore_axis_name="s")
    run = pl.kernel(
        _sc_body,
        out_type=jax.ShapeDtypeStruct((_ROWS, 8, 2, 8, _HALF), jnp.float32),
        mesh=mesh,
        scratch_types=[
            pltpu.VMEM((_MAX_SIZE * _HALF,), jnp.float32),
            pltpu.VMEM((_MAX_SIZE * _HALF,), jnp.float32),
            pltpu.VMEM((2, 2, 2, 8, _HALF), jnp.float32),
            pltpu.SemaphoreType.DMA((2,)),
        ],
    )
    out5 = run(row_flat, col_flat)
    return (
        out5.transpose(1, 3, 0, 2, 4).reshape(_BATCH, _ROWS, 2 * _HALF)
    )


kernel = sc_kernel
